# Initial kernel scaffold; baseline (speedup 1.0000x reference)
import functools

import jax
import jax.numpy as jnp
from jax import lax
from jax.experimental import pallas as pl
from jax.experimental.pallas import tpu as pltpu
from jax.experimental.pallas import tpu_sc as plsc

_FIELDS = 26
_FIELD_SIZE = 100000
_BATCH = 4096
_NC = 2
_NS = 16
_NW = _NC * _NS
_BPW = _BATCH // _NW
_CHUNK = _BPW // 16
_PER_TILE = _BPW * _FIELDS

_mesh = plsc.VectorSubcoreMesh(core_axis_name="c", subcore_axis_name="s")


@functools.partial(
    pl.kernel,
    out_type=jax.ShapeDtypeStruct((_BATCH,), jnp.float32),
    mesh=_mesh,
    scratch_types=[
        pltpu.VMEM((_PER_TILE,), jnp.int32),
        pltpu.VMEM((_PER_TILE,), jnp.int32),
        pltpu.VMEM((_PER_TILE,), jnp.float32),
        pltpu.VMEM((_BPW,), jnp.float32),
        pltpu.VMEM((16,), jnp.float32),
        pltpu.SemaphoreType.DMA,
    ],
)
def _sc_features_linear(x_hbm, fc_hbm, bias_hbm, out_hbm,
                        xv, idxv, vals, outv, bias_v, sem):
    wid = lax.axis_index("s") * _NC + lax.axis_index("c")
    base = wid * _BPW

    pltpu.sync_copy(x_hbm.at[pl.ds(base * _FIELDS, _PER_TILE)], xv)
    pltpu.sync_copy(bias_hbm, bias_v)

    lane = lax.broadcasted_iota(jnp.int32, (16,), 0)
    lane26 = lane * _FIELDS

    for f in range(_FIELDS):
        for c in range(_CHUNK):
            g = plsc.load_gather(xv, [lane26 + (c * 16 * _FIELDS + f)])
            idxv[pl.ds(f * _BPW + c * 16, 16)] = g + (f * _FIELD_SIZE)

    pltpu.async_copy(fc_hbm.at[idxv], vals, sem).wait()

    bvec = bias_v[...]
    for c in range(_CHUNK):
        acc = bvec
        for f in range(_FIELDS):
            acc = acc + vals[pl.ds(f * _BPW + c * 16, 16)]
        outv[pl.ds(c * 16, 16)] = acc

    pltpu.sync_copy(outv, out_hbm.at[pl.ds(base, _BPW)])


def kernel(x, fc, bias):
    x_flat = x.astype(jnp.int32).reshape(-1)
    fc_flat = fc.astype(jnp.float32).reshape(-1)
    bias16 = jnp.broadcast_to(bias.astype(jnp.float32), (16,))
    out = _sc_features_linear(x_flat, fc_flat, bias16)
    return out.reshape(_BATCH, 1)


# speedup vs baseline: 1.0238x; 1.0238x over previous
import functools

import jax
import jax.numpy as jnp
from jax import lax
from jax.experimental import pallas as pl
from jax.experimental.pallas import tpu as pltpu
from jax.experimental.pallas import tpu_sc as plsc

_FIELDS = 26
_FIELD_SIZE = 100000
_BATCH = 4096
_NC = 2
_NS = 16
_NW = _NC * _NS
_BPW = _BATCH // _NW
_CHUNK = _BPW // 16
_PER_TILE = _BPW * _FIELDS

_mesh = plsc.VectorSubcoreMesh(core_axis_name="c", subcore_axis_name="s")


@functools.partial(
    pl.kernel,
    out_type=jax.ShapeDtypeStruct((_BATCH,), jnp.float32),
    mesh=_mesh,
    scratch_types=[
        pltpu.VMEM((_PER_TILE,), jnp.int32),
        pltpu.VMEM((_PER_TILE,), jnp.int32),
        pltpu.VMEM((_PER_TILE,), jnp.float32),
        pltpu.VMEM((_BPW,), jnp.float32),
        pltpu.VMEM((16,), jnp.float32),
        pltpu.SemaphoreType.DMA,
    ],
    compiler_params=pltpu.CompilerParams(needs_layout_passes=False),
)
def _sc_features_linear(x_hbm, fc_hbm, bias_hbm, out_hbm,
                        xv, idxv, vals, outv, bias_v, sem):
    wid = lax.axis_index("s") * _NC + lax.axis_index("c")
    base = wid * _BPW

    pltpu.sync_copy(x_hbm.at[pl.ds(base * _FIELDS, _PER_TILE)], xv)
    pltpu.sync_copy(bias_hbm, bias_v)

    lane = lax.broadcasted_iota(jnp.int32, (16,), 0)
    lane26 = lane * _FIELDS

    for f in range(_FIELDS):
        for c in range(_CHUNK):
            g = plsc.load_gather(xv, [lane26 + (c * 16 * _FIELDS + f)])
            idxv[pl.ds(f * _BPW + c * 16, 16)] = g + (f * _FIELD_SIZE)

    pltpu.async_copy(fc_hbm.at[idxv], vals, sem).wait()

    bvec = bias_v[...]
    for c in range(_CHUNK):
        acc = bvec
        for f in range(_FIELDS):
            acc = acc + vals[pl.ds(f * _BPW + c * 16, 16)]
        outv[pl.ds(c * 16, 16)] = acc

    pltpu.sync_copy(outv, out_hbm.at[pl.ds(base, _BPW)])


def kernel(x, fc, bias):
    x_flat = x.astype(jnp.int32).reshape(-1)
    fc_flat = fc.astype(jnp.float32).reshape(-1)
    bias16 = jnp.broadcast_to(bias.astype(jnp.float32), (16,))
    out = _sc_features_linear(x_flat, fc_flat, bias16)
    return out.reshape(_BATCH, 1)


# speedup vs baseline: 1.0494x; 1.0250x over previous
import functools

import numpy as np
import jax
import jax.numpy as jnp
from jax import lax
from jax.experimental import pallas as pl
from jax.experimental.pallas import tpu as pltpu
from jax.experimental.pallas import tpu_sc as plsc

_FIELDS = 26
_FIELD_SIZE = 100000
_BATCH = 4096
_NC = 2
_NS = 16
_NW = _NC * _NS
_BPW = _BATCH // _NW
_CHUNK = _BPW // 16
_PER_TILE = _BPW * _FIELDS

_mesh = plsc.VectorSubcoreMesh(core_axis_name="c", subcore_axis_name="s")


@functools.partial(
    pl.kernel,
    out_type=jax.ShapeDtypeStruct((_BATCH,), jnp.float32),
    mesh=_mesh,
    scratch_types=[
        pltpu.VMEM((_FIELDS, _BPW), jnp.int32),
        pltpu.VMEM((_PER_TILE,), jnp.float32),
        pltpu.VMEM((_BPW,), jnp.int32),
        pltpu.VMEM((_BPW,), jnp.float32),
        pltpu.VMEM((16,), jnp.float32),
        pltpu.VMEM_SHARED((_BATCH,), jnp.float32),
        pltpu.SemaphoreType.DMA,
    ],
    compiler_params=pltpu.CompilerParams(needs_layout_passes=False),
)
def _sc_features_linear(xt_hbm, fc_hbm, bias_hbm, out_hbm,
                        xtv, vals, dstv, outv, bias_v, acc, sem):
    wid = lax.axis_index("s") * _NC + lax.axis_index("c")
    base = wid * _BPW

    pltpu.sync_copy(xt_hbm.at[:, pl.ds(base, _BPW)], xtv)
    pltpu.sync_copy(bias_hbm, bias_v)

    lane = lax.broadcasted_iota(jnp.int32, (16,), 0)
    bvec = bias_v[...]
    for c in range(_CHUNK):
        outv[pl.ds(c * 16, 16)] = bvec
        dstv[pl.ds(c * 16, 16)] = lane + (base + c * 16)
    pltpu.sync_copy(outv, acc.at[pl.ds(base, _BPW)])

    copies = [
        pltpu.async_copy(
            fc_hbm.at[pl.ds(f * _FIELD_SIZE, _FIELD_SIZE)].at[xtv.at[f]],
            vals.at[pl.ds(f * _BPW, _BPW)],
            sem,
        )
        for f in range(_FIELDS)
    ]
    for cp in copies:
        cp.wait()

    for f in range(_FIELDS):
        pltpu.sync_copy(vals.at[pl.ds(f * _BPW, _BPW)], acc.at[dstv], add=True)

    pltpu.sync_copy(acc.at[pl.ds(base, _BPW)], out_hbm.at[pl.ds(base, _BPW)])


def kernel(x, fc, bias):
    xt = x.astype(jnp.int32).T
    fc_flat = fc.astype(jnp.float32).reshape(-1)
    bias16 = jnp.broadcast_to(bias.astype(jnp.float32), (16,))
    out = _sc_features_linear(xt, fc_flat, bias16)
    return out.reshape(_BATCH, 1)


# speedup vs baseline: 4.9956x; 4.7605x over previous
import functools

import jax
import jax.numpy as jnp
from jax import lax
from jax.experimental import pallas as pl
from jax.experimental.pallas import tpu as pltpu
from jax.experimental.pallas import tpu_sc as plsc

_FIELDS = 26
_FIELD_SIZE = 100000
_BATCH = 4096
_NC = 2
_NS = 16
_NW = _NC * _NS
_BPW = _BATCH // _NW
_CHUNK = _BPW // 16
_PER_TILE = _BPW * _FIELDS

_mesh = plsc.VectorSubcoreMesh(core_axis_name="c", subcore_axis_name="s")


@functools.partial(
    pl.kernel,
    out_type=jax.ShapeDtypeStruct((_BATCH,), jnp.float32),
    mesh=_mesh,
    scratch_types=[
        pltpu.VMEM((_FIELDS, _BPW), jnp.int32),
        pltpu.VMEM((_PER_TILE,), jnp.int32),
        pltpu.VMEM((_PER_TILE,), jnp.float32),
        pltpu.VMEM((_BPW,), jnp.int32),
        pltpu.VMEM((_BPW,), jnp.float32),
        pltpu.VMEM((16,), jnp.float32),
        pltpu.VMEM_SHARED((_BATCH,), jnp.float32),
        pltpu.SemaphoreType.DMA,
    ],
    compiler_params=pltpu.CompilerParams(needs_layout_passes=False),
)
def _sc_features_linear(xt_hbm, fc_hbm, bias_hbm, out_hbm,
                        xtv, idxv, vals, dstv, outv, bias_v, acc, sem):
    wid = lax.axis_index("s") * _NC + lax.axis_index("c")
    base = wid * _BPW

    pltpu.sync_copy(xt_hbm.at[:, pl.ds(base, _BPW)], xtv)
    pltpu.sync_copy(bias_hbm, bias_v)

    lane = lax.broadcasted_iota(jnp.int32, (16,), 0)
    bvec = bias_v[...]
    for c in range(_CHUNK):
        outv[pl.ds(c * 16, 16)] = bvec
        dstv[pl.ds(c * 16, 16)] = lane + (base + c * 16)
    pltpu.sync_copy(outv, acc.at[pl.ds(base, _BPW)])

    for f in range(_FIELDS):
        for c in range(_CHUNK):
            idxv[pl.ds(f * _BPW + c * 16, 16)] = (
                xtv[f, pl.ds(c * 16, 16)] + (f * _FIELD_SIZE)
            )

    pltpu.async_copy(fc_hbm.at[0].at[idxv], vals, sem).wait()

    for f in range(_FIELDS):
        pltpu.sync_copy(vals.at[pl.ds(f * _BPW, _BPW)], acc.at[dstv], add=True)

    pltpu.sync_copy(acc.at[pl.ds(base, _BPW)], out_hbm.at[pl.ds(base, _BPW)])


def kernel(x, fc, bias):
    xt = x.astype(jnp.int32).T
    fct = fc.astype(jnp.float32).T
    bias16 = jnp.broadcast_to(bias.astype(jnp.float32), (16,))
    out = _sc_features_linear(xt, fct, bias16)
    return out.reshape(_BATCH, 1)


# speedup vs baseline: 5.8036x; 1.1618x over previous
import functools

import jax
import jax.numpy as jnp
from jax import lax
from jax.experimental import pallas as pl
from jax.experimental.pallas import tpu as pltpu
from jax.experimental.pallas import tpu_sc as plsc

_FIELDS = 26
_FIELD_SIZE = 100000
_BATCH = 4096
_NC = 2
_NS = 16
_NW = _NC * _NS
_BPW = _BATCH // _NW
_CHUNK = _BPW // 16
_PER_TILE = _BPW * _FIELDS
_F_HALF = _FIELDS // 2

_mesh = plsc.VectorSubcoreMesh(core_axis_name="c", subcore_axis_name="s")


@functools.partial(
    pl.kernel,
    out_type=jax.ShapeDtypeStruct((_BATCH,), jnp.float32),
    mesh=_mesh,
    scratch_types=[
        pltpu.VMEM((_FIELDS, _BPW), jnp.int32),
        pltpu.VMEM((_PER_TILE,), jnp.int32),
        pltpu.VMEM((_PER_TILE,), jnp.float32),
        pltpu.VMEM((_BPW,), jnp.float32),
        pltpu.VMEM((16,), jnp.float32),
        pltpu.SemaphoreType.DMA,
        pltpu.SemaphoreType.DMA,
        pltpu.SemaphoreType.DMA,
    ],
    compiler_params=pltpu.CompilerParams(needs_layout_passes=False),
)
def _sc_features_linear(xt_hbm, fc_hbm, bias_hbm, out_hbm,
                        xtv, idxv, vals, outv, bias_v, semx, sema, semb):
    wid = lax.axis_index("s") * _NC + lax.axis_index("c")
    base = wid * _BPW

    xcp = pltpu.async_copy(xt_hbm.at[:, pl.ds(base, _BPW)], xtv, semx)
    pltpu.sync_copy(bias_hbm, bias_v)
    bvec = bias_v[...]
    xcp.wait()

    half = _F_HALF * _BPW
    for f in range(_F_HALF):
        for c in range(_CHUNK):
            idxv[pl.ds(f * _BPW + c * 16, 16)] = (
                xtv[f, pl.ds(c * 16, 16)] + (f * _FIELD_SIZE)
            )
    cpa = pltpu.async_copy(fc_hbm.at[0].at[idxv.at[pl.ds(0, half)]],
                           vals.at[pl.ds(0, half)], sema)
    for f in range(_F_HALF, _FIELDS):
        for c in range(_CHUNK):
            idxv[pl.ds(f * _BPW + c * 16, 16)] = (
                xtv[f, pl.ds(c * 16, 16)] + (f * _FIELD_SIZE)
            )
    cpb = pltpu.async_copy(fc_hbm.at[0].at[idxv.at[pl.ds(half, half)]],
                           vals.at[pl.ds(half, half)], semb)

    cpa.wait()
    for c in range(_CHUNK):
        acc = bvec
        for f in range(_F_HALF):
            acc = acc + vals[pl.ds(f * _BPW + c * 16, 16)]
        outv[pl.ds(c * 16, 16)] = acc
    cpb.wait()
    for c in range(_CHUNK):
        acc = outv[pl.ds(c * 16, 16)]
        for f in range(_F_HALF, _FIELDS):
            acc = acc + vals[pl.ds(f * _BPW + c * 16, 16)]
        outv[pl.ds(c * 16, 16)] = acc

    pltpu.sync_copy(outv, out_hbm.at[pl.ds(base, _BPW)])


def kernel(x, fc, bias):
    xt = x.astype(jnp.int32).T
    fct = fc.astype(jnp.float32).T
    bias16 = jnp.broadcast_to(bias.astype(jnp.float32), (16,))
    out = _sc_features_linear(xt, fct, bias16)
    return out.reshape(_BATCH, 1)
